# baseline (speedup 1.0000x reference)
import jax
import jax.numpy as jnp
from jax.experimental import pallas as pl
from jax.experimental.pallas import tpu as pltpu

N = 2048
D = 512
H1 = 512
H2 = 512
NOUT = 128
HEADS = 4
DH = H1 // HEADS
NTYPES = 3
BLK = 256


def _fused_kernel(x_ref, t_ref, wi_ref, wv_ref, wc_ref, A_ref,
                  ac_ref, ad_ref, al_ref, w1_ref, b1_ref, w2_ref, b2_ref,
                  out_ref, haug_scr, sd_scr, sdT_scr, cs_scr):
    i = pl.program_id(0)

    @pl.when(i == 0)
    def _proj():
        xb = x_ref[...]
        h0 = jnp.dot(xb, wi_ref[...], preferred_element_type=jnp.float32)
        h1 = jnp.dot(xb, wv_ref[...], preferred_element_type=jnp.float32)
        h2 = jnp.dot(xb, wc_ref[...], preferred_element_type=jnp.float32)
        t = t_ref[...]
        h = jnp.where(t == 0, h0, jnp.where(t == 1, h1, h2))
        sd = jnp.dot(h, A_ref[...], preferred_element_type=jnp.float32)
        sd_scr[...] = sd
        sdT_scr[...] = sd.T
        cs_scr[...] = jnp.broadcast_to(
            jnp.sum(h, axis=0, keepdims=True), (8, H1))
        ones = jnp.ones((N, 1), jnp.float32)
        zpad = jnp.zeros((N, 256 - DH - 1), jnp.float32)
        for g in range(HEADS):
            haug_scr[:, g * 256:(g + 1) * 256] = jnp.concatenate(
                [h[:, g * DH:(g + 1) * DH], ones, zpad], axis=1)

    s = sd_scr[pl.ds(i * BLK, BLK), :]
    adjs = (ac_ref[...], ad_ref[...], al_ref[...])
    head_outs = []
    for hd in range(HEADS):
        h_aug = haug_scr[:, hd * 256:hd * 256 + DH + 1]
        csum = cs_scr[0:1, hd * DH:(hd + 1) * DH]
        acc = jnp.zeros((BLK, DH), jnp.float32)
        for t in range(NTYPES):
            col = t * HEADS + hd
            e = s[:, col:col + 1] + sdT_scr[12 + col, :][None, :]
            e = jnp.maximum(e, 0.01 * e)
            q = adjs[t] * jnp.exp2(e)
            nd = jnp.dot(q, h_aug, preferred_element_type=jnp.float32)
            num = nd[:, :DH] + 1e-30 * csum
            denom = nd[:, DH:DH + 1] + (N * 1e-30)
            acc = acc + num * (1.0 / denom)
        head_outs.append(acc)
    z = jnp.concatenate(head_outs, axis=1)
    z = jnp.dot(z, w1_ref[...], preferred_element_type=jnp.float32) + b1_ref[...]
    z = jnp.where(z >= 0, z, 0.1 * z)
    z = jnp.dot(z, w2_ref[...], preferred_element_type=jnp.float32) + b2_ref[...]
    out_ref[...] = jnp.where(z >= 0, z, 0.1 * z)


def kernel(x, node_types, adj_mat_control, adj_mat_data, adj_mat_call,
           W_inst, W_var, W_const, a_src, a_dst, fc1_w, fc1_b, fc2_w, fc2_b):
    nblocks = N // BLK
    types2d = node_types.astype(jnp.int32).reshape(N, 1)

    eye = jnp.eye(HEADS, dtype=jnp.float32)
    A_s = jnp.einsum('thd,hg->hdtg', a_src, eye).reshape(H1, NTYPES * HEADS)
    A_d = jnp.einsum('thd,hg->hdtg', a_dst, eye).reshape(H1, NTYPES * HEADS)
    A = jnp.pad(jnp.concatenate([A_s, A_d], axis=1), ((0, 0), (0, 128 - 24)))
    A = A * jnp.float32(1.4426950408889634)

    out = pl.pallas_call(
        _fused_kernel,
        grid=(nblocks,),
        in_specs=[
            pl.BlockSpec((N, D), lambda i: (0, 0)),
            pl.BlockSpec((N, 1), lambda i: (0, 0)),
            pl.BlockSpec((D, H1), lambda i: (0, 0)),
            pl.BlockSpec((D, H1), lambda i: (0, 0)),
            pl.BlockSpec((D, H1), lambda i: (0, 0)),
            pl.BlockSpec((H1, 128), lambda i: (0, 0)),
            pl.BlockSpec((BLK, N), lambda i: (i, 0)),
            pl.BlockSpec((BLK, N), lambda i: (i, 0)),
            pl.BlockSpec((BLK, N), lambda i: (i, 0)),
            pl.BlockSpec((H1, H2), lambda i: (0, 0)),
            pl.BlockSpec((1, H2), lambda i: (0, 0)),
            pl.BlockSpec((H2, NOUT), lambda i: (0, 0)),
            pl.BlockSpec((1, NOUT), lambda i: (0, 0)),
        ],
        out_specs=pl.BlockSpec((BLK, NOUT), lambda i: (i, 0)),
        out_shape=jax.ShapeDtypeStruct((N, NOUT), jnp.float32),
        scratch_shapes=[
            pltpu.VMEM((N, 256 * HEADS), jnp.float32),
            pltpu.VMEM((N, 128), jnp.float32),
            pltpu.VMEM((128, N), jnp.float32),
            pltpu.VMEM((8, H1), jnp.float32),
        ],
    )(x, types2d, W_inst, W_var, W_const, A,
      adj_mat_control, adj_mat_data, adj_mat_call,
      fc1_w, fc1_b.reshape(1, H2), fc2_w, fc2_b.reshape(1, NOUT))
    return out


# speedup vs baseline: 1.0025x; 1.0025x over previous
import jax
import jax.numpy as jnp
from jax.experimental import pallas as pl
from jax.experimental.pallas import tpu as pltpu

N = 2048
D = 512
H1 = 512
H2 = 512
NOUT = 128
HEADS = 4
DH = H1 // HEADS
NTYPES = 3
BLK = 256


def _fused_kernel(x_ref, t_ref, wi_ref, wv_ref, wc_ref, A_ref,
                  ac_ref, ad_ref, al_ref, w1_ref, b1_ref, w2_ref, b2_ref,
                  out_ref, h_scr, sd_scr, sdT_scr):
    i = pl.program_id(0)

    @pl.when(i == 0)
    def _proj():
        xb = x_ref[...]
        h0 = jnp.dot(xb, wi_ref[...], preferred_element_type=jnp.float32)
        h1 = jnp.dot(xb, wv_ref[...], preferred_element_type=jnp.float32)
        h2 = jnp.dot(xb, wc_ref[...], preferred_element_type=jnp.float32)
        t = t_ref[...]
        h = jnp.where(t == 0, h0, jnp.where(t == 1, h1, h2))
        h_scr[...] = h
        sd = jnp.dot(h, A_ref[...], preferred_element_type=jnp.float32)
        sd_scr[...] = sd
        sdT_scr[...] = sd.T

    hfull = h_scr[...]
    s = sd_scr[pl.ds(i * BLK, BLK), :]
    adjs = (ac_ref[...], ad_ref[...], al_ref[...])
    ones = jnp.ones((N, 1), jnp.float32)
    head_outs = []
    for hd in range(HEADS):
        h_head = hfull[:, hd * DH:(hd + 1) * DH]
        h_aug = jnp.concatenate([h_head, ones], axis=1)
        csum = jnp.sum(h_head, axis=0, keepdims=True)
        acc = jnp.zeros((BLK, DH), jnp.float32)
        for t in range(NTYPES):
            col = t * HEADS + hd
            e = s[:, col:col + 1] + sdT_scr[12 + col, :][None, :]
            e = jnp.maximum(e, 0.01 * e)
            q = adjs[t] * jnp.exp2(e)
            nd = jnp.dot(q, h_aug, preferred_element_type=jnp.float32)
            num = nd[:, :DH] + 1e-30 * csum
            denom = nd[:, DH:DH + 1] + (N * 1e-30)
            acc = acc + num * (1.0 / denom)
        head_outs.append(acc)
    z = jnp.concatenate(head_outs, axis=1)
    z = jnp.dot(z, w1_ref[...], preferred_element_type=jnp.float32) + b1_ref[...]
    z = jnp.where(z >= 0, z, 0.1 * z)
    z = jnp.dot(z, w2_ref[...], preferred_element_type=jnp.float32) + b2_ref[...]
    out_ref[...] = jnp.where(z >= 0, z, 0.1 * z)


def kernel(x, node_types, adj_mat_control, adj_mat_data, adj_mat_call,
           W_inst, W_var, W_const, a_src, a_dst, fc1_w, fc1_b, fc2_w, fc2_b):
    nblocks = N // BLK
    types2d = node_types.astype(jnp.int32).reshape(N, 1)

    eye = jnp.eye(HEADS, dtype=jnp.float32)
    A_s = jnp.einsum('thd,hg->hdtg', a_src, eye).reshape(H1, NTYPES * HEADS)
    A_d = jnp.einsum('thd,hg->hdtg', a_dst, eye).reshape(H1, NTYPES * HEADS)
    A = jnp.pad(jnp.concatenate([A_s, A_d], axis=1), ((0, 0), (0, 128 - 24)))
    A = A * jnp.float32(1.4426950408889634)

    out = pl.pallas_call(
        _fused_kernel,
        grid=(nblocks,),
        in_specs=[
            pl.BlockSpec((N, D), lambda i: (0, 0)),
            pl.BlockSpec((N, 1), lambda i: (0, 0)),
            pl.BlockSpec((D, H1), lambda i: (0, 0)),
            pl.BlockSpec((D, H1), lambda i: (0, 0)),
            pl.BlockSpec((D, H1), lambda i: (0, 0)),
            pl.BlockSpec((H1, 128), lambda i: (0, 0)),
            pl.BlockSpec((BLK, N), lambda i: (i, 0)),
            pl.BlockSpec((BLK, N), lambda i: (i, 0)),
            pl.BlockSpec((BLK, N), lambda i: (i, 0)),
            pl.BlockSpec((H1, H2), lambda i: (0, 0)),
            pl.BlockSpec((1, H2), lambda i: (0, 0)),
            pl.BlockSpec((H2, NOUT), lambda i: (0, 0)),
            pl.BlockSpec((1, NOUT), lambda i: (0, 0)),
        ],
        out_specs=pl.BlockSpec((BLK, NOUT), lambda i: (i, 0)),
        out_shape=jax.ShapeDtypeStruct((N, NOUT), jnp.float32),
        scratch_shapes=[
            pltpu.VMEM((N, H1), jnp.float32),
            pltpu.VMEM((N, 128), jnp.float32),
            pltpu.VMEM((128, N), jnp.float32),
        ],
    )(x, types2d, W_inst, W_var, W_const, A,
      adj_mat_control, adj_mat_data, adj_mat_call,
      fc1_w, fc1_b.reshape(1, H2), fc2_w, fc2_b.reshape(1, NOUT))
    return out
